# baseline (speedup 1.0000x reference)
import functools

import jax
import jax.numpy as jnp
from jax import lax
from jax.experimental import pallas as pl
from jax.experimental.pallas import tpu as pltpu
from jax.experimental.pallas import tpu_sc as plsc

N = 10000
IN_CH = 128
HID = 64
OUT_CH = 32
EPOS = 320000
ETOT = 640000
AW = 80

NC = 2
NS = 16
NW = NC * NS
STR = 624
REM0 = STR * NS
REM = N - REM0
RING = 5

EPW = EPOS // NW
BB = 128
EPW_P = 10240
NBLK = EPW_P // BB
ERING = 2
EDW = ETOT // NW
DB = 160
DH = DB // 2
NDB = EDW // DB

_mesh = plsc.VectorSubcoreMesh(core_axis_name="c", subcore_axis_name="s")
_sc_params = pltpu.CompilerParams(needs_layout_passes=False,
                                  use_tc_tiling_on_sc=False)


def _enc_body(x_ref, win_ref, bin_ref, wgat_ref, asrc_ref, adst_ref,
              h2aug_ref, aux_ref):
    t = jnp.dot(x_ref[...], win_ref[...], preferred_element_type=jnp.float32)
    t = t + bin_ref[...]
    h2 = jnp.dot(t, wgat_ref[...], preferred_element_type=jnp.float32)
    a_s = jnp.dot(h2, asrc_ref[...], preferred_element_type=jnp.float32)
    a_d = jnp.dot(h2, adst_ref[...], preferred_element_type=jnp.float32)
    r = h2.shape[0]
    h2aug_ref[...] = jnp.concatenate(
        [h2, jnp.ones((r, 1), jnp.float32), jnp.zeros((r, AW - HID - 1), jnp.float32)],
        axis=1)
    aux_ref[...] = jnp.concatenate([a_s, a_d], axis=1)


def _encode(x, w_in, b_in, w_gat, att_src, att_dst):
    grid = 10
    r = N // grid
    return pl.pallas_call(
        _enc_body,
        grid=(grid,),
        in_specs=[
            pl.BlockSpec((r, IN_CH), lambda i: (i, 0)),
            pl.BlockSpec((IN_CH, HID), lambda i: (0, 0)),
            pl.BlockSpec((1, HID), lambda i: (0, 0)),
            pl.BlockSpec((HID, HID), lambda i: (0, 0)),
            pl.BlockSpec((HID, 1), lambda i: (0, 0)),
            pl.BlockSpec((HID, 1), lambda i: (0, 0)),
        ],
        out_specs=[
            pl.BlockSpec((r, AW), lambda i: (i, 0)),
            pl.BlockSpec((r, 2), lambda i: (i, 0)),
        ],
        out_shape=[
            jax.ShapeDtypeStruct((N, AW), jnp.float32),
            jax.ShapeDtypeStruct((N, 2), jnp.float32),
        ],
    )(x, w_in, b_in, w_gat, att_src, att_dst)


@functools.partial(
    pl.kernel,
    out_type=jax.ShapeDtypeStruct((NC, N, AW), jnp.float32),
    mesh=_mesh,
    compiler_params=_sc_params,
    scratch_types=[
        pltpu.VMEM((2 * N,), jnp.float32),
        pltpu.VMEM((EPW_P,), jnp.int32),
        pltpu.VMEM((NBLK, BB), jnp.int32),
        pltpu.VMEM((EPW_P,), jnp.float32),
        pltpu.VMEM((BB, AW), jnp.float32),
        pltpu.VMEM((BB, AW), jnp.float32),
        pltpu.VMEM((16, AW), jnp.float32),
        pltpu.VMEM_SHARED((N, AW), jnp.float32),
        pltpu.SemaphoreType.DMA,
        pltpu.SemaphoreType.DMA,
        pltpu.SemaphoreType.DMA,
        pltpu.SemaphoreType.DMA,
    ],
)
def _edge_accum(h2aug_hbm, aux_hbm, srcf_hbm, dst3_hbm, out_hbm,
                aux_v, idx_s, idx_d2, wall, rb0, rb1,
                zbuf, accs, g0, g1, s0, s1):
    rows = (rb0, rb1)
    gsem = (g0, g1)
    ssem = (s0, s1)
    cid = lax.axis_index("c")
    sid = lax.axis_index("s")
    wid = sid * NC + cid
    pltpu.sync_copy(aux_hbm, aux_v)
    pltpu.sync_copy(srcf_hbm.at[wid], idx_s)
    pltpu.sync_copy(dst3_hbm.at[wid], idx_d2)

    r0_ = sid * STR

    for i in range(16):
        for v in range(AW // 16):
            zbuf[i, pl.ds(16 * v, 16)] = jnp.zeros((16,), jnp.float32)

    def _zcp(k, c):
        pltpu.sync_copy(zbuf, accs.at[pl.ds(r0_ + 16 * k, 16)])
        return c
    lax.fori_loop(0, STR // 16, _zcp, 0)

    @pl.when(sid == 0)
    def _():
        pltpu.sync_copy(zbuf.at[pl.ds(0, REM)], accs.at[pl.ds(REM0, REM)])

    def _wchunk(r, v):
        o = r * BB + 16 * v
        si = idx_s[pl.ds(o, 16)]
        di = idx_d2[r, pl.ds(16 * v, 16)]
        a = plsc.load_gather(aux_v, [si * 2])
        d = plsc.load_gather(aux_v, [di * 2 + 1])
        e = a + d
        e = jnp.where(e > 0, e, 0.2 * e)
        wall[pl.ds(o, 16)] = jnp.exp(e)

    def _wg(r, c):
        for v in range(BB // 16):
            _wchunk(r, v)
        return c
    lax.fori_loop(0, EPW // BB, _wg, 0)
    _wchunk(jnp.int32(EPW // BB), 0)
    for t in range((EPW_P - EPW) // 16):
        wall[pl.ds(EPW + 16 * t, 16)] = jnp.zeros((16,), jnp.float32)
    plsc.subcore_barrier()

    def _gather(b, j):
        pltpu.make_async_copy(
            h2aug_hbm.at[idx_s.at[pl.ds(b * BB, BB)]], rows[j], gsem[j]).start()

    for j in range(ERING - 1):
        _gather(j, j)

    def _outer(k, c):
        for j in range(ERING):
            b = ERING * k + j
            pltpu.make_async_copy(
                h2aug_hbm.at[idx_s.at[pl.ds(b * BB, BB)]], rows[j],
                gsem[j]).wait()
            wo = b * BB

            @plsc.parallel_loop(0, BB, 1, unroll=8)
            def _scale(i):
                ws = plsc.load_gather(wall, [jnp.full((16,), wo + i, jnp.int32)])
                for v in range(AW // 16):
                    rows[j][i, pl.ds(16 * v, 16)] = (
                        rows[j][i, pl.ds(16 * v, 16)] * ws)
            sc_desc = pltpu.make_async_copy(rows[j], accs.at[idx_d2.at[b]],
                                            ssem[j])
            sc_desc.start(add=True)
            sc_desc.wait()

            pj = (j - 1) % ERING
            nb = b + ERING - 1

            @pl.when(nb < NBLK)
            def _():
                _gather(nb, pj)
        return c
    lax.fori_loop(0, NBLK // ERING, _outer, 0)

    plsc.subcore_barrier()
    pltpu.sync_copy(accs.at[pl.ds(r0_, STR)], out_hbm.at[cid, pl.ds(r0_, STR)])

    @pl.when(sid == 0)
    def _():
        pltpu.sync_copy(accs.at[pl.ds(REM0, REM)], out_hbm.at[cid, pl.ds(REM0, REM)])


def _fin_body(acc_ref, aux_ref, h2aug_ref, bgat_ref, wout_ref, bout_ref, z_ref):
    acc = acc_ref[0] + acc_ref[1]
    numer = acc[:, :HID]
    denom = acc[:, HID:HID + 1]
    h2 = h2aug_ref[...][:, :HID]
    asum = aux_ref[...][:, 0:1] + aux_ref[...][:, 1:2]
    e = jnp.where(asum > 0, asum, 0.2 * asum)
    wl = jnp.exp(e)
    numer = numer + wl * h2
    denom = denom + wl
    out = numer / (denom + 1e-16) + bgat_ref[...]
    h3 = jnp.where(out > 0, out, jnp.exp(jnp.minimum(out, 0.0)) - 1.0)
    z_ref[...] = jnp.dot(h3, wout_ref[...], preferred_element_type=jnp.float32) + bout_ref[...]


def _finish(acc, aux, h2aug, b_gat, w_out, b_out):
    grid = 10
    r = N // grid
    return pl.pallas_call(
        _fin_body,
        grid=(grid,),
        in_specs=[
            pl.BlockSpec((NC, r, AW), lambda i: (0, i, 0)),
            pl.BlockSpec((r, 2), lambda i: (i, 0)),
            pl.BlockSpec((r, AW), lambda i: (i, 0)),
            pl.BlockSpec((1, HID), lambda i: (0, 0)),
            pl.BlockSpec((HID, OUT_CH), lambda i: (0, 0)),
            pl.BlockSpec((1, OUT_CH), lambda i: (0, 0)),
        ],
        out_specs=pl.BlockSpec((r, OUT_CH), lambda i: (i, 0)),
        out_shape=jax.ShapeDtypeStruct((N, OUT_CH), jnp.float32),
    )(acc, aux, h2aug, b_gat, w_out, b_out)


@functools.partial(
    pl.kernel,
    out_type=jax.ShapeDtypeStruct((ETOT,), jnp.float32),
    mesh=_mesh,
    compiler_params=_sc_params,
    scratch_types=[
        pltpu.VMEM((EDW,), jnp.int32),
        pltpu.VMEM((EDW,), jnp.int32),
        pltpu.VMEM((DB, OUT_CH), jnp.float32),
        pltpu.VMEM((DB, OUT_CH), jnp.float32),
        pltpu.VMEM((DB, OUT_CH), jnp.float32),
        pltpu.VMEM((DB, OUT_CH), jnp.float32),
        pltpu.VMEM((DB, OUT_CH), jnp.float32),
        pltpu.VMEM((DB, OUT_CH), jnp.float32),
        pltpu.VMEM((DB, OUT_CH), jnp.float32),
        pltpu.VMEM((DB, OUT_CH), jnp.float32),
        pltpu.VMEM((DB, OUT_CH), jnp.float32),
        pltpu.VMEM((DB, OUT_CH), jnp.float32),
        pltpu.VMEM((DB,), jnp.float32),
        pltpu.VMEM((DB,), jnp.float32),
        pltpu.VMEM((DB,), jnp.float32),
        pltpu.VMEM((DB,), jnp.float32),
        pltpu.VMEM((DB,), jnp.float32),
        pltpu.VMEM_SHARED((N, OUT_CH), jnp.float32),
        pltpu.SemaphoreType.DMA,
        pltpu.SemaphoreType.DMA,
        pltpu.SemaphoreType.DMA,
        pltpu.SemaphoreType.DMA,
        pltpu.SemaphoreType.DMA,
        pltpu.SemaphoreType.DMA,
        pltpu.SemaphoreType.DMA,
        pltpu.SemaphoreType.DMA,
        pltpu.SemaphoreType.DMA,
        pltpu.SemaphoreType.DMA,
    ],
)
def _decode(z_hbm, i0_hbm, i1_hbm, out_hbm,
            i0v, i1v, a0, a1, a2, a3, a4, b0, b1, b2, b3, b4,
            l0, l1, l2, l3, l4, zs, g0, g1, g2, g3, g4, o0, o1, o2, o3, o4):
    r0s = (a0, a1, a2, a3, a4)
    r1s = (b0, b1, b2, b3, b4)
    lbuf = (l0, l1, l2, l3, l4)
    gsem = (g0, g1, g2, g3, g4)
    osem = (o0, o1, o2, o3, o4)
    cid = lax.axis_index("c")
    sid = lax.axis_index("s")
    wid = sid * NC + cid
    eb = wid * EDW

    pltpu.sync_copy(i0_hbm.at[pl.ds(eb, EDW)], i0v)
    pltpu.sync_copy(i1_hbm.at[pl.ds(eb, EDW)], i1v)
    r0_ = sid * STR
    pltpu.sync_copy(z_hbm.at[pl.ds(r0_, STR)], zs.at[pl.ds(r0_, STR)])

    @pl.when(sid == 0)
    def _():
        pltpu.sync_copy(z_hbm.at[pl.ds(REM0, REM)], zs.at[pl.ds(REM0, REM)])
    plsc.subcore_barrier()

    def _gather(b, j):
        for h in range(2):
            pltpu.make_async_copy(
                zs.at[i0v.at[pl.ds(b * DB + h * DH, DH)]],
                r0s[j].at[pl.ds(h * DH, DH)], gsem[j]).start()
            pltpu.make_async_copy(
                zs.at[i1v.at[pl.ds(b * DB + h * DH, DH)]],
                r1s[j].at[pl.ds(h * DH, DH)], gsem[j]).start()

    for j in range(RING - 1):
        _gather(j, j)

    def _outer(k, c):
        for j in range(RING):
            b = RING * k + j
            for h in range(2):
                pltpu.make_async_copy(
                    zs.at[i0v.at[pl.ds(b * DB + h * DH, DH)]],
                    r0s[j].at[pl.ds(h * DH, DH)], gsem[j]).wait()
                pltpu.make_async_copy(
                    zs.at[i1v.at[pl.ds(b * DB + h * DH, DH)]],
                    r1s[j].at[pl.ds(h * DH, DH)], gsem[j]).wait()

            @pl.when(b >= RING)
            def _():
                pltpu.make_async_copy(lbuf[j], out_hbm.at[pl.ds(0, DB)],
                                      osem[j]).wait()

            for g in range(DB // 16):
                rv = lax.iota(jnp.int32, 16) + 16 * g
                parts = [jnp.zeros((16,), jnp.float32) for _ in range(4)]
                for ch in range(OUT_CH):
                    col = jnp.full((16,), ch, jnp.int32)
                    parts[ch % 4] = parts[ch % 4] + (
                        plsc.load_gather(r0s[j], [rv, col])
                        * plsc.load_gather(r1s[j], [rv, col]))
                lbuf[j][pl.ds(16 * g, 16)] = (
                    (parts[0] + parts[1]) + (parts[2] + parts[3]))
            pltpu.make_async_copy(lbuf[j], out_hbm.at[pl.ds(eb + b * DB, DB)],
                                  osem[j]).start()

            pj = (j - 1) % RING
            nb = b + RING - 1

            @pl.when(nb < NDB)
            def _():
                _gather(nb, pj)
        return c
    lax.fori_loop(0, NDB // RING, _outer, 0)

    for j in range(RING):
        pltpu.make_async_copy(lbuf[j], out_hbm.at[pl.ds(0, DB)], osem[j]).wait()


def kernel(x, pos_edge_index, neg_edge_index, W_in, b_in, W_gat, att_src,
           att_dst, b_gat, W_out, b_out):
    h2aug, aux = _encode(x, W_in, b_in.reshape(1, HID), W_gat,
                         att_src.reshape(HID, 1), att_dst.reshape(HID, 1))
    pad = jnp.zeros((NW, EPW_P - EPW), jnp.int32)
    srcp = jnp.concatenate([pos_edge_index[0].reshape(NW, EPW), pad], axis=1)
    dstp = jnp.concatenate([pos_edge_index[1].reshape(NW, EPW), pad], axis=1)
    acc = _edge_accum(h2aug, aux.reshape(2 * N), srcp,
                      dstp.reshape(NW, NBLK, BB))
    z = _finish(acc, aux, h2aug, b_gat.reshape(1, HID), W_out,
                b_out.reshape(1, OUT_CH))
    ei0 = jnp.concatenate([pos_edge_index[0], neg_edge_index[0]])
    ei1 = jnp.concatenate([pos_edge_index[1], neg_edge_index[1]])
    return _decode(z, ei0, ei1)


# speedup vs baseline: 1.1207x; 1.1207x over previous
import functools

import jax
import jax.numpy as jnp
from jax import lax
from jax.experimental import pallas as pl
from jax.experimental.pallas import tpu as pltpu
from jax.experimental.pallas import tpu_sc as plsc

N = 10000
IN_CH = 128
HID = 64
OUT_CH = 32
EPOS = 320000
ETOT = 640000
AWA = 48
AWB = 32

NC = 2
NS = 16
NW = NC * NS
STR = 624
REM0 = STR * NS
REM = N - REM0
RING = 5

EPW = EPOS // NW
BB = 80
EPW_P = 10240
NBLK = EPW_P // BB
ERING = 4
EDW = ETOT // NW
DB = 80
NDB = EDW // DB

_mesh = plsc.VectorSubcoreMesh(core_axis_name="c", subcore_axis_name="s")
_sc_params = pltpu.CompilerParams(needs_layout_passes=False,
                                  use_tc_tiling_on_sc=False)


def _enc_body(x_ref, win_ref, bin_ref, wgat_ref, asrc_ref, adst_ref,
              h2a_ref, h2b_ref, aux_ref):
    t = jnp.dot(x_ref[...], win_ref[...], preferred_element_type=jnp.float32)
    t = t + bin_ref[...]
    h2 = jnp.dot(t, wgat_ref[...], preferred_element_type=jnp.float32)
    a_s = jnp.dot(h2, asrc_ref[...], preferred_element_type=jnp.float32)
    a_d = jnp.dot(h2, adst_ref[...], preferred_element_type=jnp.float32)
    r = h2.shape[0]
    h2a_ref[...] = h2[:, :AWA]
    h2b_ref[...] = jnp.concatenate(
        [h2[:, AWA:], jnp.ones((r, 1), jnp.float32),
         jnp.zeros((r, AWB - (HID - AWA) - 1), jnp.float32)], axis=1)
    aux_ref[...] = jnp.concatenate([a_s, a_d], axis=1)


def _encode(x, w_in, b_in, w_gat, att_src, att_dst):
    grid = 10
    r = N // grid
    return pl.pallas_call(
        _enc_body,
        grid=(grid,),
        in_specs=[
            pl.BlockSpec((r, IN_CH), lambda i: (i, 0)),
            pl.BlockSpec((IN_CH, HID), lambda i: (0, 0)),
            pl.BlockSpec((1, HID), lambda i: (0, 0)),
            pl.BlockSpec((HID, HID), lambda i: (0, 0)),
            pl.BlockSpec((HID, 1), lambda i: (0, 0)),
            pl.BlockSpec((HID, 1), lambda i: (0, 0)),
        ],
        out_specs=[
            pl.BlockSpec((r, AWA), lambda i: (i, 0)),
            pl.BlockSpec((r, AWB), lambda i: (i, 0)),
            pl.BlockSpec((r, 2), lambda i: (i, 0)),
        ],
        out_shape=[
            jax.ShapeDtypeStruct((N, AWA), jnp.float32),
            jax.ShapeDtypeStruct((N, AWB), jnp.float32),
            jax.ShapeDtypeStruct((N, 2), jnp.float32),
        ],
    )(x, w_in, b_in, w_gat, att_src, att_dst)


@functools.partial(
    pl.kernel,
    out_type=[jax.ShapeDtypeStruct((NC, N, AWA), jnp.float32),
              jax.ShapeDtypeStruct((NC, N, AWB), jnp.float32)],
    mesh=_mesh,
    compiler_params=_sc_params,
    scratch_types=[
        pltpu.VMEM((2 * N,), jnp.float32),
        pltpu.VMEM((EPW_P,), jnp.int32),
        pltpu.VMEM((NBLK, BB), jnp.int32),
        pltpu.VMEM((EPW_P,), jnp.float32),
        pltpu.VMEM((BB, AWA), jnp.float32),
        pltpu.VMEM((BB, AWA), jnp.float32),
        pltpu.VMEM((BB, AWA), jnp.float32),
        pltpu.VMEM((BB, AWA), jnp.float32),
        pltpu.VMEM((BB, AWB), jnp.float32),
        pltpu.VMEM((BB, AWB), jnp.float32),
        pltpu.VMEM((BB, AWB), jnp.float32),
        pltpu.VMEM((BB, AWB), jnp.float32),
        pltpu.VMEM((16, AWA), jnp.float32),
        pltpu.VMEM((16, AWB), jnp.float32),
        pltpu.VMEM_SHARED((N, AWA), jnp.float32),
        pltpu.VMEM_SHARED((N, AWB), jnp.float32),
        pltpu.SemaphoreType.DMA,
        pltpu.SemaphoreType.DMA,
        pltpu.SemaphoreType.DMA,
        pltpu.SemaphoreType.DMA,
        pltpu.SemaphoreType.DMA,
        pltpu.SemaphoreType.DMA,
    ],
)
def _edge_accum(h2a_hbm, h2b_hbm, aux_hbm, srcf_hbm, dst3_hbm,
                outa_hbm, outb_hbm,
                aux_v, idx_s, idx_d2, wall, ra0, ra1, ra2, ra3,
                rb0, rb1, rb2, rb3, zbufa, zbufb, accsa, accsb,
                g0, g1, g2, g3, sa, sb):
    rowsa = (ra0, ra1, ra2, ra3)
    rowsb = (rb0, rb1, rb2, rb3)
    gsem = (g0, g1, g2, g3)
    cid = lax.axis_index("c")
    sid = lax.axis_index("s")
    wid = sid * NC + cid
    pltpu.sync_copy(aux_hbm, aux_v)
    pltpu.sync_copy(srcf_hbm.at[wid], idx_s)
    pltpu.sync_copy(dst3_hbm.at[wid], idx_d2)

    r0_ = sid * STR

    for i in range(16):
        for v in range(AWA // 16):
            zbufa[i, pl.ds(16 * v, 16)] = jnp.zeros((16,), jnp.float32)
        for v in range(AWB // 16):
            zbufb[i, pl.ds(16 * v, 16)] = jnp.zeros((16,), jnp.float32)

    def _zcp(k, c):
        pltpu.sync_copy(zbufa, accsa.at[pl.ds(r0_ + 16 * k, 16)])
        pltpu.sync_copy(zbufb, accsb.at[pl.ds(r0_ + 16 * k, 16)])
        return c
    lax.fori_loop(0, STR // 16, _zcp, 0)

    @pl.when(sid == 0)
    def _():
        pltpu.sync_copy(zbufa, accsa.at[pl.ds(REM0, REM)])
        pltpu.sync_copy(zbufb, accsb.at[pl.ds(REM0, REM)])

    def _wg(r, c):
        for v in range(BB // 16):
            o = r * BB + 16 * v
            si = idx_s[pl.ds(o, 16)]
            di = idx_d2[r, pl.ds(16 * v, 16)]
            a = plsc.load_gather(aux_v, [si * 2])
            d = plsc.load_gather(aux_v, [di * 2 + 1])
            e = a + d
            e = jnp.where(e > 0, e, 0.2 * e)
            wall[pl.ds(o, 16)] = jnp.exp(e)
        return c
    lax.fori_loop(0, EPW // BB, _wg, 0)
    for t in range((EPW_P - EPW) // 16):
        wall[pl.ds(EPW + 16 * t, 16)] = jnp.zeros((16,), jnp.float32)
    plsc.subcore_barrier()

    def _gather(b, j):
        pltpu.make_async_copy(
            h2a_hbm.at[idx_s.at[pl.ds(b * BB, BB)]], rowsa[j], gsem[j]).start()
        pltpu.make_async_copy(
            h2b_hbm.at[idx_s.at[pl.ds(b * BB, BB)]], rowsb[j], gsem[j]).start()

    for j in range(ERING - 1):
        _gather(j, j)

    def _outer(k, c):
        for j in range(ERING):
            b = ERING * k + j
            pltpu.make_async_copy(
                h2a_hbm.at[idx_s.at[pl.ds(b * BB, BB)]], rowsa[j],
                gsem[j]).wait()
            pltpu.make_async_copy(
                h2b_hbm.at[idx_s.at[pl.ds(b * BB, BB)]], rowsb[j],
                gsem[j]).wait()
            wo = b * BB

            @plsc.parallel_loop(0, BB, 1, unroll=8)
            def _scale(i):
                ws = plsc.load_gather(wall, [jnp.full((16,), wo + i, jnp.int32)])
                for v in range(AWA // 16):
                    rowsa[j][i, pl.ds(16 * v, 16)] = (
                        rowsa[j][i, pl.ds(16 * v, 16)] * ws)
                for v in range(AWB // 16):
                    rowsb[j][i, pl.ds(16 * v, 16)] = (
                        rowsb[j][i, pl.ds(16 * v, 16)] * ws)

            da = pltpu.make_async_copy(rowsa[j], accsa.at[idx_d2.at[b]], sa)
            db = pltpu.make_async_copy(rowsb[j], accsb.at[idx_d2.at[b]], sb)
            da.start(add=True)
            db.start(add=True)
            da.wait()
            db.wait()

            pj = (j - 1) % ERING
            nb = b + ERING - 1

            @pl.when(nb < NBLK)
            def _():
                _gather(nb, pj)
        return c
    lax.fori_loop(0, NBLK // ERING, _outer, 0)

    plsc.subcore_barrier()
    pltpu.sync_copy(accsa.at[pl.ds(r0_, STR)], outa_hbm.at[cid, pl.ds(r0_, STR)])
    pltpu.sync_copy(accsb.at[pl.ds(r0_, STR)], outb_hbm.at[cid, pl.ds(r0_, STR)])

    @pl.when(sid == 0)
    def _():
        pltpu.sync_copy(accsa.at[pl.ds(REM0, REM)], outa_hbm.at[cid, pl.ds(REM0, REM)])
        pltpu.sync_copy(accsb.at[pl.ds(REM0, REM)], outb_hbm.at[cid, pl.ds(REM0, REM)])


def _fin_body(acca_ref, accb_ref, aux_ref, h2a_ref, h2b_ref, bgat_ref,
              wout_ref, bout_ref, z_ref):
    acca = acca_ref[0] + acca_ref[1]
    accb = accb_ref[0] + accb_ref[1]
    numer = jnp.concatenate([acca, accb[:, :HID - AWA]], axis=1)
    denom = accb[:, HID - AWA:HID - AWA + 1]
    h2 = jnp.concatenate([h2a_ref[...], h2b_ref[...][:, :HID - AWA]], axis=1)
    asum = aux_ref[...][:, 0:1] + aux_ref[...][:, 1:2]
    e = jnp.where(asum > 0, asum, 0.2 * asum)
    wl = jnp.exp(e)
    numer = numer + wl * h2
    denom = denom + wl
    out = numer / (denom + 1e-16) + bgat_ref[...]
    h3 = jnp.where(out > 0, out, jnp.exp(jnp.minimum(out, 0.0)) - 1.0)
    z_ref[...] = jnp.dot(h3, wout_ref[...], preferred_element_type=jnp.float32) + bout_ref[...]


def _finish(acca, accb, aux, h2a, h2b, b_gat, w_out, b_out):
    grid = 10
    r = N // grid
    return pl.pallas_call(
        _fin_body,
        grid=(grid,),
        in_specs=[
            pl.BlockSpec((NC, r, AWA), lambda i: (0, i, 0)),
            pl.BlockSpec((NC, r, AWB), lambda i: (0, i, 0)),
            pl.BlockSpec((r, 2), lambda i: (i, 0)),
            pl.BlockSpec((r, AWA), lambda i: (i, 0)),
            pl.BlockSpec((r, AWB), lambda i: (i, 0)),
            pl.BlockSpec((1, HID), lambda i: (0, 0)),
            pl.BlockSpec((HID, OUT_CH), lambda i: (0, 0)),
            pl.BlockSpec((1, OUT_CH), lambda i: (0, 0)),
        ],
        out_specs=pl.BlockSpec((r, OUT_CH), lambda i: (i, 0)),
        out_shape=jax.ShapeDtypeStruct((N, OUT_CH), jnp.float32),
    )(acca, accb, aux, h2a, h2b, b_gat, w_out, b_out)


@functools.partial(
    pl.kernel,
    out_type=jax.ShapeDtypeStruct((ETOT,), jnp.float32),
    mesh=_mesh,
    compiler_params=_sc_params,
    scratch_types=[
        pltpu.VMEM((EDW,), jnp.int32),
        pltpu.VMEM((EDW,), jnp.int32),
        pltpu.VMEM((DB, OUT_CH), jnp.float32),
        pltpu.VMEM((DB, OUT_CH), jnp.float32),
        pltpu.VMEM((DB, OUT_CH), jnp.float32),
        pltpu.VMEM((DB, OUT_CH), jnp.float32),
        pltpu.VMEM((DB, OUT_CH), jnp.float32),
        pltpu.VMEM((DB, OUT_CH), jnp.float32),
        pltpu.VMEM((DB, OUT_CH), jnp.float32),
        pltpu.VMEM((DB, OUT_CH), jnp.float32),
        pltpu.VMEM((DB, OUT_CH), jnp.float32),
        pltpu.VMEM((DB, OUT_CH), jnp.float32),
        pltpu.VMEM((DB,), jnp.float32),
        pltpu.VMEM((DB,), jnp.float32),
        pltpu.VMEM((DB,), jnp.float32),
        pltpu.VMEM((DB,), jnp.float32),
        pltpu.VMEM((DB,), jnp.float32),
        pltpu.VMEM_SHARED((N, OUT_CH), jnp.float32),
        pltpu.SemaphoreType.DMA,
        pltpu.SemaphoreType.DMA,
        pltpu.SemaphoreType.DMA,
        pltpu.SemaphoreType.DMA,
        pltpu.SemaphoreType.DMA,
        pltpu.SemaphoreType.DMA,
        pltpu.SemaphoreType.DMA,
        pltpu.SemaphoreType.DMA,
        pltpu.SemaphoreType.DMA,
        pltpu.SemaphoreType.DMA,
    ],
)
def _decode(z_hbm, i0_hbm, i1_hbm, out_hbm,
            i0v, i1v, a0, a1, a2, a3, a4, b0, b1, b2, b3, b4,
            l0, l1, l2, l3, l4, zs, g0, g1, g2, g3, g4, o0, o1, o2, o3, o4):
    r0s = (a0, a1, a2, a3, a4)
    r1s = (b0, b1, b2, b3, b4)
    lbuf = (l0, l1, l2, l3, l4)
    gsem = (g0, g1, g2, g3, g4)
    osem = (o0, o1, o2, o3, o4)
    cid = lax.axis_index("c")
    sid = lax.axis_index("s")
    wid = sid * NC + cid
    eb = wid * EDW

    pltpu.sync_copy(i0_hbm.at[pl.ds(eb, EDW)], i0v)
    pltpu.sync_copy(i1_hbm.at[pl.ds(eb, EDW)], i1v)
    r0_ = sid * STR
    pltpu.sync_copy(z_hbm.at[pl.ds(r0_, STR)], zs.at[pl.ds(r0_, STR)])

    @pl.when(sid == 0)
    def _():
        pltpu.sync_copy(z_hbm.at[pl.ds(REM0, REM)], zs.at[pl.ds(REM0, REM)])
    plsc.subcore_barrier()

    def _gather(b, j):
        pltpu.make_async_copy(
            zs.at[i0v.at[pl.ds(b * DB, DB)]], r0s[j], gsem[j]).start()
        pltpu.make_async_copy(
            zs.at[i1v.at[pl.ds(b * DB, DB)]], r1s[j], gsem[j]).start()

    for j in range(RING - 1):
        _gather(j, j)

    def _outer(k, c):
        for j in range(RING):
            b = RING * k + j
            pltpu.make_async_copy(
                zs.at[i0v.at[pl.ds(b * DB, DB)]], r0s[j], gsem[j]).wait()
            pltpu.make_async_copy(
                zs.at[i1v.at[pl.ds(b * DB, DB)]], r1s[j], gsem[j]).wait()

            @pl.when(b >= RING)
            def _():
                pltpu.make_async_copy(lbuf[j], out_hbm.at[pl.ds(0, DB)],
                                      osem[j]).wait()

            for g in range(DB // 16):
                rv = lax.iota(jnp.int32, 16) + 16 * g
                parts = [jnp.zeros((16,), jnp.float32) for _ in range(4)]
                for ch in range(OUT_CH):
                    col = jnp.full((16,), ch, jnp.int32)
                    parts[ch % 4] = parts[ch % 4] + (
                        plsc.load_gather(r0s[j], [rv, col])
                        * plsc.load_gather(r1s[j], [rv, col]))
                lbuf[j][pl.ds(16 * g, 16)] = (
                    (parts[0] + parts[1]) + (parts[2] + parts[3]))
            pltpu.make_async_copy(lbuf[j], out_hbm.at[pl.ds(eb + b * DB, DB)],
                                  osem[j]).start()

            pj = (j - 1) % RING
            nb = b + RING - 1

            @pl.when(nb < NDB)
            def _():
                _gather(nb, pj)
        return c
    lax.fori_loop(0, NDB // RING, _outer, 0)

    for j in range(RING):
        pltpu.make_async_copy(lbuf[j], out_hbm.at[pl.ds(0, DB)], osem[j]).wait()


def kernel(x, pos_edge_index, neg_edge_index, W_in, b_in, W_gat, att_src,
           att_dst, b_gat, W_out, b_out):
    h2a, h2b, aux = _encode(x, W_in, b_in.reshape(1, HID), W_gat,
                            att_src.reshape(HID, 1), att_dst.reshape(HID, 1))
    pad = jnp.zeros((NW, EPW_P - EPW), jnp.int32)
    srcp = jnp.concatenate([pos_edge_index[0].reshape(NW, EPW), pad], axis=1)
    dstp = jnp.concatenate([pos_edge_index[1].reshape(NW, EPW), pad], axis=1)
    acca, accb = _edge_accum(h2a, h2b, aux.reshape(2 * N), srcp,
                             dstp.reshape(NW, NBLK, BB))
    z = _finish(acca, accb, aux, h2a, h2b, b_gat.reshape(1, HID), W_out,
                b_out.reshape(1, OUT_CH))
    ei0 = jnp.concatenate([pos_edge_index[0], neg_edge_index[0]])
    ei1 = jnp.concatenate([pos_edge_index[1], neg_edge_index[1]])
    return _decode(z, ei0, ei1)
